# baseline (speedup 1.0000x reference)
import jax
import jax.numpy as jnp
from jax.experimental import pallas as pl
from jax.experimental.pallas import tpu as pltpu

B, C, L = 32, 256, 2048
D4 = 192
E = 16
K = 2

CHUNK = 2
NCHUNK = B // CHUNK
NBUF = 4
N = CHUNK * C


def _route_chunk(xb, w1, b1, w2, b2):
    h = jnp.maximum(
        jnp.dot(xb, w1, preferred_element_type=jnp.float32) + b1, 0.0
    )
    logits = jnp.dot(h, w2, preferred_element_type=jnp.float32) + b2
    lt = logits.T

    m = jnp.max(lt, axis=0, keepdims=True)
    ex = jnp.exp(lt - m)
    p = ex / jnp.sum(ex, axis=0, keepdims=True)

    lane = jax.lax.broadcasted_iota(jnp.int32, (E, N), 0)
    p1 = jnp.max(p, axis=0, keepdims=True)
    i1 = jnp.min(jnp.where(p == p1, lane, E), axis=0, keepdims=True)
    pm = jnp.where(lane == i1, -jnp.inf, p)
    p2 = jnp.max(pm, axis=0, keepdims=True)
    i2 = jnp.min(jnp.where(pm == p2, lane, E), axis=0, keepdims=True)

    s = p1 + p2 + 1e-6
    return jnp.where(lane == i1, p1 / s, 0.0) + jnp.where(lane == i2, p2 / s, 0.0)


def _fused_kernel(x_hbm, w1_ref, b1_ref, w2_ref, b2_ref, out_ref,
                  buf, gbuf, sems):
    w1 = w1_ref[...]
    b1 = b1_ref[...]
    w2 = w2_ref[...]
    b2 = b2_ref[...]

    for s in range(NBUF):
        pltpu.make_async_copy(
            x_hbm.at[pl.ds(s * CHUNK, CHUNK)], buf.at[s], sems.at[s]
        ).start()

    for i in range(NCHUNK):
        s = i % NBUF
        pltpu.make_async_copy(
            x_hbm.at[pl.ds(i * CHUNK, CHUNK)], buf.at[s], sems.at[s]
        ).wait()
        xb = buf[s].reshape(N, L)
        g = _route_chunk(xb, w1, b1, w2, b2)
        nxt = i + NBUF
        if nxt < NCHUNK:
            pltpu.make_async_copy(
                x_hbm.at[pl.ds(nxt * CHUNK, CHUNK)], buf.at[s], sems.at[s]
            ).start()
        gbuf[:, i * N:(i + 1) * N] = g

    gf = gbuf[...].reshape(E * B, C)
    out_ref[...] = jax.lax.dot_general(
        gf, gf, (((0,), (0,)), ((), ())), preferred_element_type=jnp.float32
    ) * (1.0 / B)


@jax.jit
def kernel(x, gate_w1, gate_b1, gate_w2, gate_b2, expert_w, expert_b):
    del expert_w, expert_b
    b1 = gate_b1.reshape(1, D4)
    b2 = gate_b2.reshape(1, E)
    vmem = pl.BlockSpec(memory_space=pltpu.MemorySpace.VMEM)
    return pl.pallas_call(
        _fused_kernel,
        in_specs=[
            pl.BlockSpec(memory_space=pltpu.MemorySpace.HBM),
            vmem, vmem, vmem, vmem,
        ],
        out_specs=pl.BlockSpec(memory_space=pltpu.MemorySpace.VMEM),
        out_shape=jax.ShapeDtypeStruct((C, C), jnp.float32),
        scratch_shapes=[
            pltpu.VMEM((NBUF, CHUNK, C, L), jnp.float32),
            pltpu.VMEM((E, B * C), jnp.float32),
            pltpu.SemaphoreType.DMA((NBUF,)),
        ],
    )(x, gate_w1, b1, gate_w2, b2)


# speedup vs baseline: 1.0584x; 1.0584x over previous
import jax
import jax.numpy as jnp
from jax.experimental import pallas as pl
from jax.experimental.pallas import tpu as pltpu

B, C, L = 32, 256, 2048
D4 = 192
E = 16
K = 2

CHUNK = 4
NCHUNK = B // CHUNK
NBUF = 4
N = CHUNK * C


def _route_chunk(xb, w1, b1, w2, b2):
    h = jnp.maximum(
        jnp.dot(xb, w1, preferred_element_type=jnp.float32) + b1, 0.0
    )
    logits = jnp.dot(h, w2, preferred_element_type=jnp.float32) + b2
    lt = logits.T

    m = jnp.max(lt, axis=0, keepdims=True)
    ex = jnp.exp(lt - m)
    p = ex / jnp.sum(ex, axis=0, keepdims=True)

    lane = jax.lax.broadcasted_iota(jnp.int32, (E, N), 0)
    p1 = jnp.max(p, axis=0, keepdims=True)
    i1 = jnp.min(jnp.where(p == p1, lane, E), axis=0, keepdims=True)
    pm = jnp.where(lane == i1, -jnp.inf, p)
    p2 = jnp.max(pm, axis=0, keepdims=True)
    i2 = jnp.min(jnp.where(pm == p2, lane, E), axis=0, keepdims=True)

    s = p1 + p2 + 1e-6
    return jnp.where(lane == i1, p1 / s, 0.0) + jnp.where(lane == i2, p2 / s, 0.0)


def _fused_kernel(x_hbm, w1_ref, b1_ref, w2_ref, b2_ref, out_ref,
                  buf, gbuf, sems):
    w1 = w1_ref[...]
    b1 = b1_ref[...]
    w2 = w2_ref[...]
    b2 = b2_ref[...]

    for s in range(NBUF):
        pltpu.make_async_copy(
            x_hbm.at[pl.ds(s * CHUNK, CHUNK)], buf.at[s], sems.at[s]
        ).start()

    for i in range(NCHUNK):
        s = i % NBUF
        pltpu.make_async_copy(
            x_hbm.at[pl.ds(i * CHUNK, CHUNK)], buf.at[s], sems.at[s]
        ).wait()
        xb = buf[s].reshape(N, L)
        g = _route_chunk(xb, w1, b1, w2, b2)
        nxt = i + NBUF
        if nxt < NCHUNK:
            pltpu.make_async_copy(
                x_hbm.at[pl.ds(nxt * CHUNK, CHUNK)], buf.at[s], sems.at[s]
            ).start()
        gbuf[:, i * N:(i + 1) * N] = g

    gf = gbuf[...].reshape(E * B, C)
    out_ref[...] = jax.lax.dot_general(
        gf, gf, (((0,), (0,)), ((), ())), preferred_element_type=jnp.float32
    ) * (1.0 / B)


@jax.jit
def kernel(x, gate_w1, gate_b1, gate_w2, gate_b2, expert_w, expert_b):
    del expert_w, expert_b
    b1 = gate_b1.reshape(1, D4)
    b2 = gate_b2.reshape(1, E)
    vmem = pl.BlockSpec(memory_space=pltpu.MemorySpace.VMEM)
    return pl.pallas_call(
        _fused_kernel,
        in_specs=[
            pl.BlockSpec(memory_space=pltpu.MemorySpace.HBM),
            vmem, vmem, vmem, vmem,
        ],
        out_specs=pl.BlockSpec(memory_space=pltpu.MemorySpace.VMEM),
        out_shape=jax.ShapeDtypeStruct((C, C), jnp.float32),
        scratch_shapes=[
            pltpu.VMEM((NBUF, CHUNK, C, L), jnp.float32),
            pltpu.VMEM((E, B * C), jnp.float32),
            pltpu.SemaphoreType.DMA((NBUF,)),
        ],
    )(x, gate_w1, b1, gate_w2, b2)


# speedup vs baseline: 1.1220x; 1.0601x over previous
import jax
import jax.numpy as jnp
from jax.experimental import pallas as pl
from jax.experimental.pallas import tpu as pltpu

B, C, L = 32, 256, 2048
D4 = 192
E = 16
K = 2

BPB = 8
N = BPB * C
NSTEP = B // BPB


def _fused_kernel(x_ref, w1_ref, b1_ref, w2_ref, b2_ref, out_ref, gbuf):
    step = pl.program_id(0)
    xb = x_ref[...].reshape(N, L)
    h = jnp.maximum(
        jnp.dot(xb, w1_ref[...], preferred_element_type=jnp.float32) + b1_ref[...],
        0.0,
    )
    logits = jnp.dot(h, w2_ref[...], preferred_element_type=jnp.float32) + b2_ref[...]
    lt = logits.T

    m = jnp.max(lt, axis=0, keepdims=True)
    ex = jnp.exp(lt - m)
    p = ex / jnp.sum(ex, axis=0, keepdims=True)

    lane = jax.lax.broadcasted_iota(jnp.int32, (E, N), 0)
    p1 = jnp.max(p, axis=0, keepdims=True)
    i1 = jnp.min(jnp.where(p == p1, lane, E), axis=0, keepdims=True)
    pm = jnp.where(lane == i1, -jnp.inf, p)
    p2 = jnp.max(pm, axis=0, keepdims=True)
    i2 = jnp.min(jnp.where(pm == p2, lane, E), axis=0, keepdims=True)

    s = p1 + p2 + 1e-6
    g = jnp.where(lane == i1, p1 / s, 0.0) + jnp.where(lane == i2, p2 / s, 0.0)

    gbuf[:, pl.ds(step * N, N)] = g

    @pl.when(step == NSTEP - 1)
    def _final():
        gf = gbuf[...].reshape(E * B, C)
        out_ref[...] = jax.lax.dot_general(
            gf, gf, (((0,), (0,)), ((), ())), preferred_element_type=jnp.float32
        ) * (1.0 / B)


@jax.jit
def kernel(x, gate_w1, gate_b1, gate_w2, gate_b2, expert_w, expert_b):
    del expert_w, expert_b
    b1 = gate_b1.reshape(1, D4)
    b2 = gate_b2.reshape(1, E)
    return pl.pallas_call(
        _fused_kernel,
        grid=(NSTEP,),
        in_specs=[
            pl.BlockSpec((BPB, C, L), lambda b: (b, 0, 0)),
            pl.BlockSpec((L, D4), lambda b: (0, 0)),
            pl.BlockSpec((1, D4), lambda b: (0, 0)),
            pl.BlockSpec((D4, E), lambda b: (0, 0)),
            pl.BlockSpec((1, E), lambda b: (0, 0)),
        ],
        out_specs=pl.BlockSpec((C, C), lambda b: (0, 0)),
        out_shape=jax.ShapeDtypeStruct((C, C), jnp.float32),
        scratch_shapes=[
            pltpu.VMEM((E, B * C), jnp.float32),
        ],
        compiler_params=pltpu.CompilerParams(
            dimension_semantics=("arbitrary",),
        ),
    )(x, gate_w1, b1, gate_w2, b2)


# speedup vs baseline: 1.1273x; 1.0048x over previous
import jax
import jax.numpy as jnp
from jax.experimental import pallas as pl
from jax.experimental.pallas import tpu as pltpu

B, C, L = 32, 256, 2048
D4 = 192
E = 16
K = 2

BPB = 4
N = BPB * C
NSTEP = B // BPB


def _fused_kernel(x_ref, w1_ref, b1_ref, w2_ref, b2_ref, out_ref, gbuf):
    step = pl.program_id(0)
    xb = x_ref[...].reshape(N, L)
    h = jnp.maximum(
        jnp.dot(xb, w1_ref[...], preferred_element_type=jnp.float32) + b1_ref[...],
        0.0,
    )
    logits = jnp.dot(h, w2_ref[...], preferred_element_type=jnp.float32) + b2_ref[...]
    lt = logits.T

    m = jnp.max(lt, axis=0, keepdims=True)
    ex = jnp.exp(lt - m)
    p = ex / jnp.sum(ex, axis=0, keepdims=True)

    lane = jax.lax.broadcasted_iota(jnp.int32, (E, N), 0)
    p1 = jnp.max(p, axis=0, keepdims=True)
    i1 = jnp.min(jnp.where(p == p1, lane, E), axis=0, keepdims=True)
    pm = jnp.where(lane == i1, -jnp.inf, p)
    p2 = jnp.max(pm, axis=0, keepdims=True)
    i2 = jnp.min(jnp.where(pm == p2, lane, E), axis=0, keepdims=True)

    s = p1 + p2 + 1e-6
    g = jnp.where(lane == i1, p1 / s, 0.0) + jnp.where(lane == i2, p2 / s, 0.0)

    gbuf[:, pl.ds(step * N, N)] = g

    @pl.when(step == NSTEP - 1)
    def _final():
        gf = gbuf[...].reshape(E * B, C)
        out_ref[...] = jax.lax.dot_general(
            gf, gf, (((0,), (0,)), ((), ())), preferred_element_type=jnp.float32
        ) * (1.0 / B)


@jax.jit
def kernel(x, gate_w1, gate_b1, gate_w2, gate_b2, expert_w, expert_b):
    del expert_w, expert_b
    b1 = gate_b1.reshape(1, D4)
    b2 = gate_b2.reshape(1, E)
    return pl.pallas_call(
        _fused_kernel,
        grid=(NSTEP,),
        in_specs=[
            pl.BlockSpec((BPB, C, L), lambda b: (b, 0, 0)),
            pl.BlockSpec((L, D4), lambda b: (0, 0)),
            pl.BlockSpec((1, D4), lambda b: (0, 0)),
            pl.BlockSpec((D4, E), lambda b: (0, 0)),
            pl.BlockSpec((1, E), lambda b: (0, 0)),
        ],
        out_specs=pl.BlockSpec((C, C), lambda b: (0, 0)),
        out_shape=jax.ShapeDtypeStruct((C, C), jnp.float32),
        scratch_shapes=[
            pltpu.VMEM((E, B * C), jnp.float32),
        ],
        compiler_params=pltpu.CompilerParams(
            dimension_semantics=("arbitrary",),
        ),
    )(x, gate_w1, b1, gate_w2, b2)
